# Initial kernel scaffold; baseline (speedup 1.0000x reference)
import jax
import jax.numpy as jnp
from jax.experimental import pallas as pl


def kernel(x, edge_index, edge_type, R_diagonal):
    raise NotImplementedError("write your pallas kernel here")


# speedup vs baseline: 5.8680x; 5.8680x over previous
import functools

import jax
import jax.numpy as jnp
from jax import lax
from jax.experimental import pallas as pl
from jax.experimental.pallas import tpu as pltpu
from jax.experimental.pallas import tpu_sc as plsc


_NW = 32
_K = 80
_LANES = 16


def _normalize_body(x_ref, xn_ref):
    xv = x_ref[...]
    n = jnp.sqrt(jnp.sum(xv * xv, axis=1, keepdims=True))
    xn_ref[...] = xv / jnp.maximum(n, 1e-12)


_PAD = 17


def _score_body(n_ch, e_w, xn_hbm, r_hbm, src_hbm, dst_hbm, et_hbm, out_hbm,
                src_v, dst_v, et_v, s_buf, o_buf, r_buf, tmp_v, out_v, sem):
    wid = lax.axis_index("s") * 2 + lax.axis_index("c")
    base = wid * e_w
    pltpu.sync_copy(src_hbm.at[pl.ds(base, e_w)], src_v)
    pltpu.sync_copy(dst_hbm.at[pl.ds(base, e_w)], dst_v)
    pltpu.sync_copy(et_hbm.at[pl.ds(base, e_w)], et_v)
    n_chunks = e_w // _K
    lane17 = lax.iota(jnp.int32, _LANES) * _PAD

    @pl.loop(0, n_chunks)
    def _chunk(i):
        off = i * _K
        c1 = pltpu.async_copy(xn_hbm.at[src_v.at[pl.ds(off, _K)]], s_buf, sem)
        c2 = pltpu.async_copy(xn_hbm.at[dst_v.at[pl.ds(off, _K)]], o_buf, sem)
        c3 = pltpu.async_copy(r_hbm.at[et_v.at[pl.ds(off, _K)]], r_buf, sem)
        c1.wait()
        c2.wait()
        c3.wait()

        @pl.loop(0, _K // _LANES)
        def _group(g):
            for l in range(_LANES):
                e = g * _LANES + l
                acc = (s_buf[e, pl.ds(0, _LANES)]
                       * o_buf[e, pl.ds(0, _LANES)]
                       * r_buf[e, pl.ds(0, _LANES)])
                for j in range(1, n_ch // _LANES):
                    acc = acc + (s_buf[e, pl.ds(j * _LANES, _LANES)]
                                 * o_buf[e, pl.ds(j * _LANES, _LANES)]
                                 * r_buf[e, pl.ds(j * _LANES, _LANES)])
                tmp_v[pl.ds(l * _PAD, _LANES)] = acc
            scores = plsc.load_gather(tmp_v, [lane17])
            for k in range(1, _LANES):
                scores = scores + plsc.load_gather(tmp_v, [lane17 + k])
            out_v[pl.ds(off + g * _LANES, _LANES)] = scores

    pltpu.sync_copy(out_v, out_hbm.at[pl.ds(base, e_w)])


def kernel(x, edge_index, edge_type, R_diagonal):
    n_nodes, n_ch = x.shape
    n_rel = R_diagonal.shape[0]
    n_edges = edge_index.shape[1]

    xn = pl.pallas_call(
        _normalize_body,
        out_shape=jax.ShapeDtypeStruct((n_nodes, n_ch), jnp.float32),
    )(x)

    src = edge_index[0].astype(jnp.int32)
    dst = edge_index[1].astype(jnp.int32)
    et = edge_type.astype(jnp.int32)

    e_w = n_edges // _NW
    mesh = plsc.VectorSubcoreMesh(core_axis_name="c", subcore_axis_name="s")
    score = pl.kernel(
        functools.partial(_score_body, n_ch, e_w),
        out_type=jax.ShapeDtypeStruct((n_edges,), jnp.float32),
        mesh=mesh,
        compiler_params=pltpu.CompilerParams(needs_layout_passes=False),
        scratch_types=[
            pltpu.VMEM((e_w,), jnp.int32),
            pltpu.VMEM((e_w,), jnp.int32),
            pltpu.VMEM((e_w,), jnp.int32),
            pltpu.VMEM((_K, n_ch), jnp.float32),
            pltpu.VMEM((_K, n_ch), jnp.float32),
            pltpu.VMEM((_K, n_ch), jnp.float32),
            pltpu.VMEM((_LANES * _PAD,), jnp.float32),
            pltpu.VMEM((e_w,), jnp.float32),
            pltpu.SemaphoreType.DMA,
        ],
    )(xn, R_diagonal, src, dst, et)
    return score


# speedup vs baseline: 7.0308x; 1.1982x over previous
import functools

import jax
import jax.numpy as jnp
from jax import lax
from jax.experimental import pallas as pl
from jax.experimental.pallas import tpu as pltpu
from jax.experimental.pallas import tpu_sc as plsc


_NW = 32
_K = 80
_LANES = 16
_PAD = 17


def _prep_body(x_ref, r_ref, xn_ref, rb_ref):
    xv = x_ref[...]
    n = jnp.sqrt(jnp.sum(xv * xv, axis=1, keepdims=True))
    xn_ref[...] = (xv / jnp.maximum(n, 1e-12)).astype(jnp.bfloat16)
    rb_ref[...] = r_ref[...].astype(jnp.bfloat16)


def _dot3(s_buf, o_buf, r_buf, e, j):
    out = []
    for buf in (s_buf, o_buf, r_buf):
        w = buf[e, pl.ds(j * _LANES, _LANES)]
        b = plsc.bitcast(w, jnp.bfloat16)
        out.append(plsc.unpack(b, format=plsc.PackFormat.INTERLEAVED,
                               preferred_element_type=jnp.float32))
    (sa, sb), (oa, ob), (ra, rb) = out
    return sa * oa * ra + sb * ob * rb


def _score_body(n_w, e_w, xn_hbm, r_hbm, src_hbm, dst_hbm, et_hbm, out_hbm,
                src_v, dst_v, et_v, s_buf, o_buf, r_buf, tmp_v, out_v, sem):
    wid = lax.axis_index("s") * 2 + lax.axis_index("c")
    base = wid * e_w
    pltpu.sync_copy(src_hbm.at[pl.ds(base, e_w)], src_v)
    pltpu.sync_copy(dst_hbm.at[pl.ds(base, e_w)], dst_v)
    pltpu.sync_copy(et_hbm.at[pl.ds(base, e_w)], et_v)
    n_chunks = e_w // _K
    lane17 = lax.iota(jnp.int32, _LANES) * _PAD

    @pl.loop(0, n_chunks)
    def _chunk(i):
        off = i * _K
        c1 = pltpu.async_copy(xn_hbm.at[src_v.at[pl.ds(off, _K)]], s_buf, sem)
        c2 = pltpu.async_copy(xn_hbm.at[dst_v.at[pl.ds(off, _K)]], o_buf, sem)
        c3 = pltpu.async_copy(r_hbm.at[et_v.at[pl.ds(off, _K)]], r_buf, sem)
        c1.wait()
        c2.wait()
        c3.wait()

        @pl.loop(0, _K // _LANES)
        def _group(g):
            for l in range(_LANES):
                e = g * _LANES + l
                acc = _dot3(s_buf, o_buf, r_buf, e, 0)
                for j in range(1, n_w // _LANES):
                    acc = acc + _dot3(s_buf, o_buf, r_buf, e, j)
                tmp_v[pl.ds(l * _PAD, _LANES)] = acc
            scores = plsc.load_gather(tmp_v, [lane17])
            for k in range(1, _LANES):
                scores = scores + plsc.load_gather(tmp_v, [lane17 + k])
            out_v[pl.ds(off + g * _LANES, _LANES)] = scores

    pltpu.sync_copy(out_v, out_hbm.at[pl.ds(base, e_w)])


def kernel(x, edge_index, edge_type, R_diagonal):
    n_nodes, n_ch = x.shape
    n_rel = R_diagonal.shape[0]
    n_edges = edge_index.shape[1]
    n_w = n_ch // 2

    xn_b, r_b = pl.pallas_call(
        _prep_body,
        out_shape=(
            jax.ShapeDtypeStruct((n_nodes, n_ch), jnp.bfloat16),
            jax.ShapeDtypeStruct((n_rel, n_ch), jnp.bfloat16),
        ),
    )(x, R_diagonal)

    xn_w = lax.bitcast_convert_type(
        xn_b.reshape(n_nodes, n_w, 2), jnp.int32)
    r_w = lax.bitcast_convert_type(
        r_b.reshape(n_rel, n_w, 2), jnp.int32)

    src = edge_index[0].astype(jnp.int32)
    dst = edge_index[1].astype(jnp.int32)
    et = edge_type.astype(jnp.int32)

    e_w = n_edges // _NW
    mesh = plsc.VectorSubcoreMesh(core_axis_name="c", subcore_axis_name="s")
    score = pl.kernel(
        functools.partial(_score_body, n_w, e_w),
        out_type=jax.ShapeDtypeStruct((n_edges,), jnp.float32),
        mesh=mesh,
        compiler_params=pltpu.CompilerParams(needs_layout_passes=False,
                                             use_tc_tiling_on_sc=False),
        scratch_types=[
            pltpu.VMEM((e_w,), jnp.int32),
            pltpu.VMEM((e_w,), jnp.int32),
            pltpu.VMEM((e_w,), jnp.int32),
            pltpu.VMEM((_K, n_w), jnp.int32),
            pltpu.VMEM((_K, n_w), jnp.int32),
            pltpu.VMEM((_K, n_w), jnp.int32),
            pltpu.VMEM((_LANES * _PAD,), jnp.float32),
            pltpu.VMEM((e_w,), jnp.float32),
            pltpu.SemaphoreType.DMA,
        ],
    )(xn_w, r_w, src, dst, et)
    return score


# speedup vs baseline: 11.1872x; 1.5912x over previous
import functools

import jax
import jax.numpy as jnp
from jax import lax
from jax.experimental import pallas as pl
from jax.experimental.pallas import tpu as pltpu
from jax.experimental.pallas import tpu_sc as plsc


_NW = 32
_K = 80
_LANES = 16
_PAD = 17


def _prep_body(x_ref, r_ref, xn_ref, rb_ref):
    xv = x_ref[...]
    n = jnp.sqrt(jnp.sum(xv * xv, axis=1, keepdims=True))
    xn_ref[...] = (xv / jnp.maximum(n, 1e-12)).astype(jnp.bfloat16)
    rb_ref[...] = r_ref[...].astype(jnp.bfloat16)


def _dot3(s_buf, o_buf, r_buf, e, j):
    out = []
    for buf in (s_buf, o_buf, r_buf):
        w = buf[e, pl.ds(j * _LANES, _LANES)]
        b = plsc.bitcast(w, jnp.bfloat16)
        out.append(plsc.unpack(b, format=plsc.PackFormat.INTERLEAVED,
                               preferred_element_type=jnp.float32))
    (sa, sb), (oa, ob), (ra, rb) = out
    return sa * oa * ra + sb * ob * rb


def _score_body(n_w, e_w, xn_hbm, r_hbm, src_hbm, dst_hbm, et_hbm, out_hbm,
                src_v, dst_v, et_v, sA, oA, rA, sB, oB, rB, tmp_v, out_v,
                semA, semB):
    wid = lax.axis_index("s") * 2 + lax.axis_index("c")
    base = wid * e_w
    pltpu.sync_copy(src_hbm.at[pl.ds(base, e_w)], src_v)
    pltpu.sync_copy(dst_hbm.at[pl.ds(base, e_w)], dst_v)
    pltpu.sync_copy(et_hbm.at[pl.ds(base, e_w)], et_v)
    n_chunks = e_w // _K
    lane17 = lax.iota(jnp.int32, _LANES) * _PAD
    bufs = {0: (sA, oA, rA, semA), 1: (sB, oB, rB, semB)}

    def copies(i, p):
        s_buf, o_buf, r_buf, sem = bufs[p]
        off = i * _K
        return (
            pltpu.make_async_copy(
                xn_hbm.at[src_v.at[pl.ds(off, _K)]], s_buf, sem),
            pltpu.make_async_copy(
                xn_hbm.at[dst_v.at[pl.ds(off, _K)]], o_buf, sem),
            pltpu.make_async_copy(
                r_hbm.at[et_v.at[pl.ds(off, _K)]], r_buf, sem),
        )

    def compute(i, p):
        s_buf, o_buf, r_buf, _ = bufs[p]
        off = i * _K

        @pl.loop(0, _K // _LANES)
        def _group(g):
            for l in range(_LANES):
                e = g * _LANES + l
                acc = _dot3(s_buf, o_buf, r_buf, e, 0)
                for j in range(1, n_w // _LANES):
                    acc = acc + _dot3(s_buf, o_buf, r_buf, e, j)
                tmp_v[pl.ds(l * _PAD, _LANES)] = acc
            scores = plsc.load_gather(tmp_v, [lane17])
            for k in range(1, _LANES):
                scores = scores + plsc.load_gather(tmp_v, [lane17 + k])
            out_v[pl.ds(off + g * _LANES, _LANES)] = scores

    for c in copies(0, 0):
        c.start()

    @pl.loop(0, (n_chunks + 1) // 2)
    def _pair(h):
        i0 = 2 * h
        i1 = i0 + 1

        @pl.when(i1 < n_chunks)
        def _():
            for c in copies(i1, 1):
                c.start()

        for c in copies(i0, 0):
            c.wait()
        compute(i0, 0)

        @pl.when(i1 < n_chunks)
        def _():
            @pl.when(i1 + 1 < n_chunks)
            def _():
                for c in copies(i1 + 1, 0):
                    c.start()

            for c in copies(i1, 1):
                c.wait()
            compute(i1, 1)

    pltpu.sync_copy(out_v, out_hbm.at[pl.ds(base, e_w)])


def kernel(x, edge_index, edge_type, R_diagonal):
    n_nodes, n_ch = x.shape
    n_rel = R_diagonal.shape[0]
    n_edges = edge_index.shape[1]
    n_w = n_ch // 2

    xn_b, r_b = pl.pallas_call(
        _prep_body,
        out_shape=(
            jax.ShapeDtypeStruct((n_nodes, n_ch), jnp.bfloat16),
            jax.ShapeDtypeStruct((n_rel, n_ch), jnp.bfloat16),
        ),
    )(x, R_diagonal)

    xn_w = lax.bitcast_convert_type(
        xn_b.reshape(n_nodes, n_w, 2), jnp.int32)
    r_w = lax.bitcast_convert_type(
        r_b.reshape(n_rel, n_w, 2), jnp.int32)

    src = edge_index[0].astype(jnp.int32)
    dst = edge_index[1].astype(jnp.int32)
    et = edge_type.astype(jnp.int32)

    e_w = n_edges // _NW
    mesh = plsc.VectorSubcoreMesh(core_axis_name="c", subcore_axis_name="s")
    score = pl.kernel(
        functools.partial(_score_body, n_w, e_w),
        out_type=jax.ShapeDtypeStruct((n_edges,), jnp.float32),
        mesh=mesh,
        compiler_params=pltpu.CompilerParams(needs_layout_passes=False,
                                             use_tc_tiling_on_sc=False),
        scratch_types=[
            pltpu.VMEM((e_w,), jnp.int32),
            pltpu.VMEM((e_w,), jnp.int32),
            pltpu.VMEM((e_w,), jnp.int32),
            pltpu.VMEM((_K, n_w), jnp.int32),
            pltpu.VMEM((_K, n_w), jnp.int32),
            pltpu.VMEM((_K, n_w), jnp.int32),
            pltpu.VMEM((_K, n_w), jnp.int32),
            pltpu.VMEM((_K, n_w), jnp.int32),
            pltpu.VMEM((_K, n_w), jnp.int32),
            pltpu.VMEM((_LANES * _PAD,), jnp.float32),
            pltpu.VMEM((e_w,), jnp.float32),
            pltpu.SemaphoreType.DMA,
            pltpu.SemaphoreType.DMA,
        ],
    )(xn_w, r_w, src, dst, et)
    return score


# speedup vs baseline: 11.8950x; 1.0633x over previous
import functools

import jax
import jax.numpy as jnp
from jax import lax
from jax.experimental import pallas as pl
from jax.experimental.pallas import tpu as pltpu
from jax.experimental.pallas import tpu_sc as plsc


_NW = 32
_K = 80
_LANES = 16
_PAD = 17


def _prep_body(x_ref, r_ref, xn_ref, rb_ref):
    xv = x_ref[...]
    n = jnp.sqrt(jnp.sum(xv * xv, axis=1, keepdims=True))
    xn_ref[...] = (xv / jnp.maximum(n, 1e-12)).astype(jnp.bfloat16)
    rb_ref[...] = r_ref[...].astype(jnp.bfloat16)


def _dot3(s_buf, o_buf, r_buf, e, j):
    sb, ob, rb = (
        plsc.bitcast(buf[e, pl.ds(j * _LANES, _LANES)], jnp.bfloat16)
        for buf in (s_buf, o_buf, r_buf))
    ta, tb = plsc.unpack(sb * ob * rb, format=plsc.PackFormat.INTERLEAVED,
                         preferred_element_type=jnp.float32)
    return ta + tb


def _score_body(n_w, e_w, xn_hbm, r_hbm, src_hbm, dst_hbm, et_hbm, out_hbm,
                src_v, dst_v, et_v, sA, oA, rA, sB, oB, rB, tmp_v, out_v,
                semA, semB):
    wid = lax.axis_index("s") * 2 + lax.axis_index("c")
    base = wid * e_w
    pltpu.sync_copy(src_hbm.at[pl.ds(base, e_w)], src_v)
    pltpu.sync_copy(dst_hbm.at[pl.ds(base, e_w)], dst_v)
    pltpu.sync_copy(et_hbm.at[pl.ds(base, e_w)], et_v)
    n_chunks = e_w // _K
    lane17 = lax.iota(jnp.int32, _LANES) * _PAD
    bufs = {0: (sA, oA, rA, semA), 1: (sB, oB, rB, semB)}

    def copies(i, p):
        s_buf, o_buf, r_buf, sem = bufs[p]
        off = i * _K
        return (
            pltpu.make_async_copy(
                xn_hbm.at[src_v.at[pl.ds(off, _K)]], s_buf, sem),
            pltpu.make_async_copy(
                xn_hbm.at[dst_v.at[pl.ds(off, _K)]], o_buf, sem),
            pltpu.make_async_copy(
                r_hbm.at[et_v.at[pl.ds(off, _K)]], r_buf, sem),
        )

    def compute(i, p):
        s_buf, o_buf, r_buf, _ = bufs[p]
        off = i * _K

        @pl.loop(0, _K // _LANES)
        def _group(g):
            for l in range(_LANES):
                e = g * _LANES + l
                acc = _dot3(s_buf, o_buf, r_buf, e, 0)
                for j in range(1, n_w // _LANES):
                    acc = acc + _dot3(s_buf, o_buf, r_buf, e, j)
                tmp_v[pl.ds(l * _PAD, _LANES)] = acc
            scores = plsc.load_gather(tmp_v, [lane17])
            for k in range(1, _LANES):
                scores = scores + plsc.load_gather(tmp_v, [lane17 + k])
            out_v[pl.ds(off + g * _LANES, _LANES)] = scores

    for c in copies(0, 0):
        c.start()

    @pl.loop(0, (n_chunks + 1) // 2)
    def _pair(h):
        i0 = 2 * h
        i1 = i0 + 1

        @pl.when(i1 < n_chunks)
        def _():
            for c in copies(i1, 1):
                c.start()

        for c in copies(i0, 0):
            c.wait()
        compute(i0, 0)

        @pl.when(i1 < n_chunks)
        def _():
            @pl.when(i1 + 1 < n_chunks)
            def _():
                for c in copies(i1 + 1, 0):
                    c.start()

            for c in copies(i1, 1):
                c.wait()
            compute(i1, 1)

    pltpu.sync_copy(out_v, out_hbm.at[pl.ds(base, e_w)])


def kernel(x, edge_index, edge_type, R_diagonal):
    n_nodes, n_ch = x.shape
    n_rel = R_diagonal.shape[0]
    n_edges = edge_index.shape[1]
    n_w = n_ch // 2

    xn_b, r_b = pl.pallas_call(
        _prep_body,
        out_shape=(
            jax.ShapeDtypeStruct((n_nodes, n_ch), jnp.bfloat16),
            jax.ShapeDtypeStruct((n_rel, n_ch), jnp.bfloat16),
        ),
    )(x, R_diagonal)

    xn_w = lax.bitcast_convert_type(
        xn_b.reshape(n_nodes, n_w, 2), jnp.int32)
    r_w = lax.bitcast_convert_type(
        r_b.reshape(n_rel, n_w, 2), jnp.int32)

    src = edge_index[0].astype(jnp.int32)
    dst = edge_index[1].astype(jnp.int32)
    et = edge_type.astype(jnp.int32)

    e_w = n_edges // _NW
    mesh = plsc.VectorSubcoreMesh(core_axis_name="c", subcore_axis_name="s")
    score = pl.kernel(
        functools.partial(_score_body, n_w, e_w),
        out_type=jax.ShapeDtypeStruct((n_edges,), jnp.float32),
        mesh=mesh,
        compiler_params=pltpu.CompilerParams(needs_layout_passes=False,
                                             use_tc_tiling_on_sc=False),
        scratch_types=[
            pltpu.VMEM((e_w,), jnp.int32),
            pltpu.VMEM((e_w,), jnp.int32),
            pltpu.VMEM((e_w,), jnp.int32),
            pltpu.VMEM((_K, n_w), jnp.int32),
            pltpu.VMEM((_K, n_w), jnp.int32),
            pltpu.VMEM((_K, n_w), jnp.int32),
            pltpu.VMEM((_K, n_w), jnp.int32),
            pltpu.VMEM((_K, n_w), jnp.int32),
            pltpu.VMEM((_K, n_w), jnp.int32),
            pltpu.VMEM((_LANES * _PAD,), jnp.float32),
            pltpu.VMEM((e_w,), jnp.float32),
            pltpu.SemaphoreType.DMA,
            pltpu.SemaphoreType.DMA,
        ],
    )(xn_w, r_w, src, dst, et)
    return score


# speedup vs baseline: 14.0464x; 1.1809x over previous
import functools

import jax
import jax.numpy as jnp
from jax import lax
from jax.experimental import pallas as pl
from jax.experimental.pallas import tpu as pltpu
from jax.experimental.pallas import tpu_sc as plsc


_NW = 32
_K = 80
_LANES = 16
_PAD = 17


def _pack_rows(y):
    w = y.shape[1] // 2
    u = lax.bitcast_convert_type(y.astype(jnp.bfloat16), jnp.uint16)
    lo = u[:, :w].astype(jnp.uint32)
    hi = u[:, w:].astype(jnp.uint32) << 16
    return lax.bitcast_convert_type(lo | hi, jnp.int32)


def _prep_body(x_ref, r_ref, xn_ref, rb_ref):
    xv = x_ref[...]
    n = jnp.sqrt(jnp.sum(xv * xv, axis=1, keepdims=True))
    xn_ref[...] = _pack_rows(xv / jnp.maximum(n, 1e-12))
    rb_ref[...] = _pack_rows(r_ref[...])


def _dot3(s_buf, o_buf, r_buf, e, j):
    sb, ob, rb = (
        plsc.bitcast(buf[e, pl.ds(j * _LANES, _LANES)], jnp.bfloat16)
        for buf in (s_buf, o_buf, r_buf))
    ta, tb = plsc.unpack(sb * ob * rb, format=plsc.PackFormat.INTERLEAVED,
                         preferred_element_type=jnp.float32)
    return ta + tb


def _score_body(n_w, e_w, xn_hbm, r_hbm, ei_hbm, et_hbm, out_hbm,
                src_v, dst_v, et_v, sA, oA, rA, sB, oB, rB, tmp_v, out_v,
                semA, semB):
    wid = lax.axis_index("s") * 2 + lax.axis_index("c")
    base = wid * e_w
    pltpu.sync_copy(ei_hbm.at[0, pl.ds(base, e_w)], src_v)
    pltpu.sync_copy(ei_hbm.at[1, pl.ds(base, e_w)], dst_v)
    pltpu.sync_copy(et_hbm.at[pl.ds(base, e_w)], et_v)
    n_chunks = e_w // _K
    lane17 = lax.iota(jnp.int32, _LANES) * _PAD
    bufs = {0: (sA, oA, rA, semA), 1: (sB, oB, rB, semB)}

    def copies(i, p):
        s_buf, o_buf, r_buf, sem = bufs[p]
        off = i * _K
        return (
            pltpu.make_async_copy(
                xn_hbm.at[src_v.at[pl.ds(off, _K)]], s_buf, sem),
            pltpu.make_async_copy(
                xn_hbm.at[dst_v.at[pl.ds(off, _K)]], o_buf, sem),
            pltpu.make_async_copy(
                r_hbm.at[et_v.at[pl.ds(off, _K)]], r_buf, sem),
        )

    def compute(i, p):
        s_buf, o_buf, r_buf, _ = bufs[p]
        off = i * _K

        @pl.loop(0, _K // _LANES)
        def _group(g):
            for l in range(_LANES):
                e = g * _LANES + l
                acc = _dot3(s_buf, o_buf, r_buf, e, 0)
                for j in range(1, n_w // _LANES):
                    acc = acc + _dot3(s_buf, o_buf, r_buf, e, j)
                tmp_v[pl.ds(l * _PAD, _LANES)] = acc
            scores = plsc.load_gather(tmp_v, [lane17])
            for k in range(1, _LANES):
                scores = scores + plsc.load_gather(tmp_v, [lane17 + k])
            out_v[pl.ds(off + g * _LANES, _LANES)] = scores

    for c in copies(0, 0):
        c.start()

    @pl.loop(0, (n_chunks + 1) // 2)
    def _pair(h):
        i0 = 2 * h
        i1 = i0 + 1

        @pl.when(i1 < n_chunks)
        def _():
            for c in copies(i1, 1):
                c.start()

        for c in copies(i0, 0):
            c.wait()
        compute(i0, 0)

        @pl.when(i1 < n_chunks)
        def _():
            @pl.when(i1 + 1 < n_chunks)
            def _():
                for c in copies(i1 + 1, 0):
                    c.start()

            for c in copies(i1, 1):
                c.wait()
            compute(i1, 1)

    pltpu.sync_copy(out_v, out_hbm.at[pl.ds(base, e_w)])


def kernel(x, edge_index, edge_type, R_diagonal):
    n_nodes, n_ch = x.shape
    n_rel = R_diagonal.shape[0]
    n_edges = edge_index.shape[1]
    n_w = n_ch // 2

    xn_w, r_w = pl.pallas_call(
        _prep_body,
        out_shape=(
            jax.ShapeDtypeStruct((n_nodes, n_w), jnp.int32),
            jax.ShapeDtypeStruct((n_rel, n_w), jnp.int32),
        ),
    )(x, R_diagonal)

    ei = edge_index.astype(jnp.int32)
    et = edge_type.astype(jnp.int32)

    e_w = n_edges // _NW
    mesh = plsc.VectorSubcoreMesh(core_axis_name="c", subcore_axis_name="s")
    score = pl.kernel(
        functools.partial(_score_body, n_w, e_w),
        out_type=jax.ShapeDtypeStruct((n_edges,), jnp.float32),
        mesh=mesh,
        compiler_params=pltpu.CompilerParams(needs_layout_passes=False,
                                             use_tc_tiling_on_sc=False),
        scratch_types=[
            pltpu.VMEM((e_w,), jnp.int32),
            pltpu.VMEM((e_w,), jnp.int32),
            pltpu.VMEM((e_w,), jnp.int32),
            pltpu.VMEM((_K, n_w), jnp.int32),
            pltpu.VMEM((_K, n_w), jnp.int32),
            pltpu.VMEM((_K, n_w), jnp.int32),
            pltpu.VMEM((_K, n_w), jnp.int32),
            pltpu.VMEM((_K, n_w), jnp.int32),
            pltpu.VMEM((_K, n_w), jnp.int32),
            pltpu.VMEM((_LANES * _PAD,), jnp.float32),
            pltpu.VMEM((e_w,), jnp.float32),
            pltpu.SemaphoreType.DMA,
            pltpu.SemaphoreType.DMA,
        ],
    )(xn_w, r_w, ei, et)
    return score
